# baseline (speedup 1.0000x reference)
import functools

import jax
import jax.numpy as jnp
from jax import lax
from jax.experimental import pallas as pl
from jax.experimental.pallas import tpu as pltpu
from jax.experimental.pallas import tpu_sc as plsc


def _make_sc_gather(V, D, N, B, NC, NS, chunk):
    NW = NC * NS
    n_per_w = N // NW
    n_ch = n_per_w // chunk
    Bh = B // 2
    mesh = plsc.VectorSubcoreMesh(core_axis_name="c", subcore_axis_name="s")

    @functools.partial(
        pl.kernel,
        mesh=mesh,
        compiler_params=pltpu.CompilerParams(use_tc_tiling_on_sc=False),
        out_type=jax.ShapeDtypeStruct((N // 2, 2 * D), jnp.float32),
        scratch_types=[
            pltpu.VMEM((n_ch, chunk), jnp.int32),
            pltpu.VMEM((chunk, D), jnp.float32),
            pltpu.VMEM((chunk, D), jnp.float32),
            pltpu.SemaphoreType.DMA,
            pltpu.SemaphoreType.DMA,
        ],
    )
    def gather_k(table_hbm, idx_hbm, out_hbm, idx_v, buf0, buf1, sem0, sem1):
        wid = lax.axis_index("s") * NC + lax.axis_index("c")
        pltpu.sync_copy(idx_hbm.at[wid], idx_v)
        base = wid * n_per_w

        def start(j, buf, sem):
            pltpu.make_async_copy(table_hbm.at[idx_v.at[j]], buf, sem).start()

        def drain(j, buf, sem):
            pltpu.make_async_copy(table_hbm.at[idx_v.at[j]], buf, sem).wait()
            k0 = base + j * chunk
            b = lax.rem(k0, B)
            r = (k0 // B) * Bh + lax.rem(b, Bh)
            h = (b // Bh) * D
            pltpu.sync_copy(buf, out_hbm.at[pl.ds(r, chunk), pl.ds(h, D)])

        start(0, buf0, sem0)

        def body(p, _):
            j = 2 * p
            start(j + 1, buf1, sem1)
            drain(j, buf0, sem0)
            start(j + 2, buf0, sem0)
            drain(j + 1, buf1, sem1)
            return 0

        lax.fori_loop(0, (n_ch - 1) // 2, body, 0)
        if n_ch % 2 == 1:
            drain(n_ch - 1, buf0, sem0)
        else:
            start(n_ch - 1, buf1, sem1)
            drain(n_ch - 2, buf0, sem0)
            drain(n_ch - 1, buf1, sem1)

    return gather_k


def _x_proj(Hd, wih, x2):
    xe = x2[:, 0:Hd]
    xo = x2[:, Hd:2 * Hd]
    dn = (((1,), (1,)), ((), ()))
    ge = lax.dot_general(wih, xe, dn, preferred_element_type=jnp.float32)
    go = lax.dot_general(wih, xo, dn, preferred_element_type=jnp.float32)
    return jnp.concatenate([ge, go], axis=1)


def _lstm_body(Hd, T, has_alias, x0_ref, xn_ref, h0_ref, c0_ref, wih_ref,
               whh_ref, b_ref, *refs):
    if has_alias:
        refs = refs[1:]
    out_ref, hT_ref, cT_ref, h_s, c_s, gx_s = refs
    t = pl.program_id(0)
    wih = wih_ref[...]

    @pl.when(t == 0)
    def _():
        h_s[...] = h0_ref[...]
        c_s[...] = c0_ref[...]
        gx_s[...] = _x_proj(Hd, wih, x0_ref[0])

    h = h_s[...]
    gates = gx_s[...] + jnp.dot(whh_ref[...], h,
                                preferred_element_type=jnp.float32)
    gates = gates + b_ref[...]
    gx_s[...] = _x_proj(Hd, wih, xn_ref[0])
    i = jax.nn.sigmoid(gates[0 * Hd:1 * Hd, :])
    f = jax.nn.sigmoid(gates[1 * Hd:2 * Hd, :])
    g = jnp.tanh(gates[2 * Hd:3 * Hd, :])
    o = jax.nn.sigmoid(gates[3 * Hd:4 * Hd, :])
    c = f * c_s[...] + i * g
    h_new = o * jnp.tanh(c)
    h_s[...] = h_new
    c_s[...] = c
    out_ref[0] = h_new

    @pl.when(t == T - 1)
    def _():
        hT_ref[...] = h_new
        cT_ref[...] = c


def _lstm_call(x2, h0t, c0t, wih, whh, bias, t_total, t_off=0, out_alias=None,
               interpret=False):
    T, Bh, Hd2 = x2.shape
    Hd = Hd2 // 2
    B = Bh * 2
    out_shapes = (
        jax.ShapeDtypeStruct((t_total, Hd, B), jnp.float32),
        jax.ShapeDtypeStruct((Hd, B), jnp.float32),
        jax.ShapeDtypeStruct((Hd, B), jnp.float32),
    )
    in_specs = [
        pl.BlockSpec((1, Bh, Hd2), lambda t: (0, 0, 0)),
        pl.BlockSpec((1, Bh, Hd2),
                     lambda t: (jnp.minimum(t + 1, T - 1), 0, 0)),
        pl.BlockSpec((Hd, B), lambda t: (0, 0)),
        pl.BlockSpec((Hd, B), lambda t: (0, 0)),
        pl.BlockSpec((4 * Hd, Hd), lambda t: (0, 0)),
        pl.BlockSpec((4 * Hd, Hd), lambda t: (0, 0)),
        pl.BlockSpec((4 * Hd, 1), lambda t: (0, 0)),
    ]
    args = [x2, x2, h0t, c0t, wih, whh, bias]
    aliases = {}
    body = functools.partial(_lstm_body, Hd, T, out_alias is not None)
    if out_alias is not None:
        in_specs.append(pl.BlockSpec(memory_space=pl.ANY))
        args.append(out_alias)
        aliases = {7: 0}
    return pl.pallas_call(
        body,
        grid=(T,),
        in_specs=in_specs,
        out_specs=(
            pl.BlockSpec((1, Hd, B), lambda t: (t + t_off, 0, 0)),
            pl.BlockSpec((Hd, B), lambda t: (0, 0)),
            pl.BlockSpec((Hd, B), lambda t: (0, 0)),
        ),
        out_shape=out_shapes,
        scratch_shapes=[
            pltpu.VMEM((Hd, B), jnp.float32),
            pltpu.VMEM((Hd, B), jnp.float32),
            pltpu.VMEM((4 * Hd, B), jnp.float32),
        ],
        input_output_aliases=aliases,
        interpret=interpret,
    )(*args)


def kernel(input_src, h0, c0, embed, W_ih, W_hh, b_ih, b_hh):
    T, B = input_src.shape
    V, Hd = embed.shape
    Bh = B // 2

    info = plsc.get_sparse_core_info()
    NC, NS = info.num_cores, info.num_subcores
    NW = NC * NS

    parts = [8, 14, 14, 14]
    chunk = 64

    idx = input_src.astype(jnp.int32)
    h0t = h0[0].T
    c0t = c0[0].T
    bias = (b_ih + b_hh).reshape(4 * Hd, 1)

    t0 = 0
    out_t = None
    ht, ct = h0t, c0t
    for Tp in parts:
        Np = Tp * B
        idx_p = idx[t0:t0 + Tp].reshape(NW, Np // (NW * chunk), chunk)
        emb_p = _make_sc_gather(V, Hd, Np, B, NC, NS, chunk)(embed, idx_p)
        x2p = emb_p.reshape(Tp, Bh, 2 * Hd)
        out_t, ht, ct = _lstm_call(x2p, ht, ct, W_ih, W_hh, bias, T,
                                   t_off=t0, out_alias=out_t)
        t0 += Tp
    hTt, cTt = ht, ct

    out = out_t.transpose(0, 2, 1)
    hT = hTt.T[None]
    cT = cTt.T[None]
    return out, (hT, cT)


# speedup vs baseline: 1.0258x; 1.0258x over previous
import functools

import jax
import jax.numpy as jnp
from jax import lax
from jax.experimental import pallas as pl
from jax.experimental.pallas import tpu as pltpu
from jax.experimental.pallas import tpu_sc as plsc


def _make_sc_gather(V, D, N, B, NC, NS, chunk):
    NW = NC * NS
    n_per_w = N // NW
    n_ch = n_per_w // chunk
    Bh = B // 2
    mesh = plsc.VectorSubcoreMesh(core_axis_name="c", subcore_axis_name="s")

    @functools.partial(
        pl.kernel,
        mesh=mesh,
        compiler_params=pltpu.CompilerParams(use_tc_tiling_on_sc=False),
        out_type=jax.ShapeDtypeStruct((N // 2, 2 * D), jnp.float32),
        scratch_types=[
            pltpu.VMEM((n_ch, chunk), jnp.int32),
            pltpu.VMEM((chunk, D), jnp.float32),
            pltpu.VMEM((chunk, D), jnp.float32),
            pltpu.SemaphoreType.DMA,
            pltpu.SemaphoreType.DMA,
        ],
    )
    def gather_k(table_hbm, idx_hbm, out_hbm, idx_v, buf0, buf1, sem0, sem1):
        wid = lax.axis_index("s") * NC + lax.axis_index("c")
        pltpu.sync_copy(idx_hbm.at[wid], idx_v)
        base = wid * n_per_w

        def start(j, buf, sem):
            pltpu.make_async_copy(table_hbm.at[idx_v.at[j]], buf, sem).start()

        def drain(j, buf, sem):
            pltpu.make_async_copy(table_hbm.at[idx_v.at[j]], buf, sem).wait()
            k0 = base + j * chunk
            b = lax.rem(k0, B)
            r = (k0 // B) * Bh + lax.rem(b, Bh)
            h = (b // Bh) * D
            pltpu.sync_copy(buf, out_hbm.at[pl.ds(r, chunk), pl.ds(h, D)])

        start(0, buf0, sem0)

        def body(p, _):
            j = 2 * p
            start(j + 1, buf1, sem1)
            drain(j, buf0, sem0)
            start(j + 2, buf0, sem0)
            drain(j + 1, buf1, sem1)
            return 0

        lax.fori_loop(0, (n_ch - 1) // 2, body, 0)
        if n_ch % 2 == 1:
            drain(n_ch - 1, buf0, sem0)
        else:
            start(n_ch - 1, buf1, sem1)
            drain(n_ch - 2, buf0, sem0)
            drain(n_ch - 1, buf1, sem1)

    return gather_k


def _x_proj(Hd, wih, x2):
    xe = x2[:, 0:Hd]
    xo = x2[:, Hd:2 * Hd]
    dn = (((1,), (1,)), ((), ()))
    ge = lax.dot_general(wih, xe, dn, preferred_element_type=jnp.float32)
    go = lax.dot_general(wih, xo, dn, preferred_element_type=jnp.float32)
    return jnp.concatenate([ge, go], axis=1)


def _lstm_body(Hd, T, has_alias, x0_ref, xn_ref, h0_ref, c0_ref, wih_ref,
               whh_ref, b_ref, *refs):
    if has_alias:
        refs = refs[1:]
    out_ref, hT_ref, cT_ref, h_s, c_s, gx_s = refs
    t = pl.program_id(0)
    wih = wih_ref[...]

    @pl.when(t == 0)
    def _():
        h_s[...] = h0_ref[...]
        c_s[...] = c0_ref[...]
        gx_s[...] = _x_proj(Hd, wih, x0_ref[0])

    h = h_s[...]
    gates = gx_s[...] + jnp.dot(whh_ref[...], h,
                                preferred_element_type=jnp.float32)
    gates = gates + b_ref[...]
    gx_s[...] = _x_proj(Hd, wih, xn_ref[0])
    i = jax.nn.sigmoid(gates[0 * Hd:1 * Hd, :])
    f = jax.nn.sigmoid(gates[1 * Hd:2 * Hd, :])
    g = jnp.tanh(gates[2 * Hd:3 * Hd, :])
    o = jax.nn.sigmoid(gates[3 * Hd:4 * Hd, :])
    c = f * c_s[...] + i * g
    h_new = o * jnp.tanh(c)
    h_s[...] = h_new
    c_s[...] = c
    out_ref[0] = h_new

    @pl.when(t == T - 1)
    def _():
        hT_ref[...] = h_new
        cT_ref[...] = c


def _lstm_call(x2, h0t, c0t, wih, whh, bias, t_total, t_off=0, out_alias=None,
               interpret=False):
    T, Bh, Hd2 = x2.shape
    Hd = Hd2 // 2
    B = Bh * 2
    out_shapes = (
        jax.ShapeDtypeStruct((t_total, Hd, B), jnp.float32),
        jax.ShapeDtypeStruct((Hd, B), jnp.float32),
        jax.ShapeDtypeStruct((Hd, B), jnp.float32),
    )
    in_specs = [
        pl.BlockSpec((1, Bh, Hd2), lambda t: (0, 0, 0)),
        pl.BlockSpec((1, Bh, Hd2),
                     lambda t: (jnp.minimum(t + 1, T - 1), 0, 0)),
        pl.BlockSpec((Hd, B), lambda t: (0, 0)),
        pl.BlockSpec((Hd, B), lambda t: (0, 0)),
        pl.BlockSpec((4 * Hd, Hd), lambda t: (0, 0)),
        pl.BlockSpec((4 * Hd, Hd), lambda t: (0, 0)),
        pl.BlockSpec((4 * Hd, 1), lambda t: (0, 0)),
    ]
    args = [x2, x2, h0t, c0t, wih, whh, bias]
    aliases = {}
    body = functools.partial(_lstm_body, Hd, T, out_alias is not None)
    if out_alias is not None:
        in_specs.append(pl.BlockSpec(memory_space=pl.ANY))
        args.append(out_alias)
        aliases = {7: 0}
    return pl.pallas_call(
        body,
        grid=(T,),
        in_specs=in_specs,
        out_specs=(
            pl.BlockSpec((1, Hd, B), lambda t: (t + t_off, 0, 0)),
            pl.BlockSpec((Hd, B), lambda t: (0, 0)),
            pl.BlockSpec((Hd, B), lambda t: (0, 0)),
        ),
        out_shape=out_shapes,
        scratch_shapes=[
            pltpu.VMEM((Hd, B), jnp.float32),
            pltpu.VMEM((Hd, B), jnp.float32),
            pltpu.VMEM((4 * Hd, B), jnp.float32),
        ],
        input_output_aliases=aliases,
        interpret=interpret,
    )(*args)


def kernel(input_src, h0, c0, embed, W_ih, W_hh, b_ih, b_hh):
    T, B = input_src.shape
    V, Hd = embed.shape
    Bh = B // 2

    info = plsc.get_sparse_core_info()
    NC, NS = info.num_cores, info.num_subcores
    NW = NC * NS

    parts = [16, 34]
    chunk = 64

    idx = input_src.astype(jnp.int32)
    h0t = h0[0].T
    c0t = c0[0].T
    bias = (b_ih + b_hh).reshape(4 * Hd, 1)

    t0 = 0
    out_t = None
    ht, ct = h0t, c0t
    for Tp in parts:
        Np = Tp * B
        idx_p = idx[t0:t0 + Tp].reshape(NW, Np // (NW * chunk), chunk)
        emb_p = _make_sc_gather(V, Hd, Np, B, NC, NS, chunk)(embed, idx_p)
        x2p = emb_p.reshape(Tp, Bh, 2 * Hd)
        out_t, ht, ct = _lstm_call(x2p, ht, ct, W_ih, W_hh, bias, T,
                                   t_off=t0, out_alias=out_t)
        t0 += Tp
    hTt, cTt = ht, ct

    out = out_t.transpose(0, 2, 1)
    hT = hTt.T[None]
    cT = cTt.T[None]
    return out, (hT, cT)
